# baseline (speedup 1.0000x reference)
import numpy as np
import jax
import jax.numpy as jnp
from jax.experimental import pallas as pl
from jax.experimental.pallas import tpu as pltpu


def _pick_block(n):
    for bm in (1000, 512, 400, 256, 200, 128, 80, 40, 16, 8):
        if n % bm == 0:
            return bm
    return 1


def _make_body(nb, bsz):
    nmain = nb * nb

    def body(ii, jj, oo, x_ref, adj_ref, w0_ref, w1_ref, w2_ref, b_ref,
             out_ref, t1_ref, t2a_ref, p_ref):
        g = pl.program_id(0)
        i = ii[g]
        j = jj[g]
        rows_i = pl.ds(i * bsz, bsz)
        rows_j = pl.ds(j * bsz, bsz)
        adj_blk = adj_ref[...].reshape(bsz, bsz)

        @pl.when(g < nmain)
        def _main_sweep():
            t1c = jnp.dot(adj_blk, x_ref[rows_j, :],
                          preferred_element_type=jnp.float32)

            @pl.when(j == 0)
            def _row_start():
                t1_ref[rows_i, :] = t1c
                t2a_ref[rows_i, :] = jnp.zeros_like(t1c)
                p_ref[rows_i, :] = jnp.dot(
                    x_ref[rows_i, :], w0_ref[...] - w2_ref[...],
                    preferred_element_type=jnp.float32) + b_ref[...]

            @pl.when(j != 0)
            def _row_acc():
                t1_ref[rows_i, :] += t1c

            @pl.when(j < i)
            def _dual_serve():
                t2a_ref[rows_i, :] += jnp.dot(
                    adj_blk, t1_ref[rows_j, :],
                    preferred_element_type=jnp.float32)

        @pl.when(g >= nmain)
        def _triangle_sweep():
            t2a_ref[rows_i, :] += jnp.dot(
                adj_blk, t1_ref[rows_j, :], preferred_element_type=jnp.float32)

            @pl.when(j == nb - 1)
            def _finalize_row():
                out_ref[...] = (
                    p_ref[rows_i, :]
                    + jnp.dot(t1_ref[rows_i, :], w1_ref[...],
                              preferred_element_type=jnp.float32)
                    + jnp.dot(t2a_ref[rows_i, :], 2.0 * w2_ref[...],
                              preferred_element_type=jnp.float32)
                )

    return body


def kernel(x, adj, W0, W1, W2, b):
    n, d_in = x.shape
    d_out = W0.shape[1]
    bsz = _pick_block(n)
    nb = n // bsz
    b2d = b.reshape(1, d_out).astype(jnp.float32)

    ii, jj, oo = [], [], []
    for i in range(nb):
        for j in range(nb):
            ii.append(i)
            jj.append(j)
            oo.append(0)
    for i in range(nb):
        for j in range(i, nb):
            ii.append(i)
            jj.append(j)
            oo.append(i)
    ii = jnp.asarray(np.array(ii, dtype=np.int32))
    jj = jnp.asarray(np.array(jj, dtype=np.int32))
    oo = jnp.asarray(np.array(oo, dtype=np.int32))

    adj4 = adj.reshape(n, nb, 1, bsz)

    grid_spec = pltpu.PrefetchScalarGridSpec(
        num_scalar_prefetch=3,
        grid=(nb * nb + nb * (nb + 1) // 2,),
        in_specs=[
            pl.BlockSpec((n, d_in), lambda g, a, c, o: (0, 0)),
            pl.BlockSpec((bsz, 1, 1, bsz),
                         lambda g, a, c, o: (a[g], c[g], 0, 0)),
            pl.BlockSpec((d_in, d_out), lambda g, a, c, o: (0, 0)),
            pl.BlockSpec((d_in, d_out), lambda g, a, c, o: (0, 0)),
            pl.BlockSpec((d_in, d_out), lambda g, a, c, o: (0, 0)),
            pl.BlockSpec((1, d_out), lambda g, a, c, o: (0, 0)),
        ],
        out_specs=pl.BlockSpec((bsz, d_out), lambda g, a, c, o: (o[g], 0)),
        scratch_shapes=[
            pltpu.VMEM((n, d_in), jnp.float32),
            pltpu.VMEM((n, d_out), jnp.float32),
            pltpu.VMEM((n, d_out), jnp.float32),
        ],
    )
    out = pl.pallas_call(
        _make_body(nb, bsz),
        grid_spec=grid_spec,
        out_shape=jax.ShapeDtypeStruct((n, d_out), jnp.float32),
        compiler_params=pltpu.CompilerParams(
            dimension_semantics=("arbitrary",),
            vmem_limit_bytes=100 * 1024 * 1024,
        ),
    )(ii, jj, oo, x, adj4, W0, W1, W2, b2d)
    return out


# speedup vs baseline: 8.1701x; 8.1701x over previous
import numpy as np
import jax
import jax.numpy as jnp
from jax.experimental import pallas as pl
from jax.experimental.pallas import tpu as pltpu

_CW = 1024


def _pick_block(n):
    for bm in (200, 128, 80, 40, 16, 8):
        if n % bm == 0:
            return bm
    return 1


def _make_body(nb, bsz, n, nc, lw, nres, cw):
    nmain = nb

    def body(ai, bi, bc, oo, x_ref, adja_ref, adjb_ref,
             w0_ref, w1_ref, w2_ref, b_ref,
             out_ref, t1_ref, t2a_ref, p_ref):
        g = pl.program_id(0)

        @pl.when(g < nmain)
        def _main_sweep():
            i = g
            rows_i = pl.ds(i * bsz, bsz)
            stripe = adja_ref[...]
            t1c = jnp.dot(stripe, x_ref[...],
                          preferred_element_type=jnp.float32)
            t1_ref[rows_i, :] = t1c
            t2a_ref[rows_i, :] = jnp.zeros_like(t1c)
            p_ref[rows_i, :] = (
                jnp.dot(x_ref[rows_i, :], w0_ref[...] - w2_ref[...],
                        preferred_element_type=jnp.float32)
                + jnp.dot(t1c, w1_ref[...],
                          preferred_element_type=jnp.float32)
                + b_ref[...]
            )
            for c in range(nc - 1):
                @pl.when(cw * (c + 1) <= bsz * i)
                def _dual_serve(c=c):
                    t2a_ref[rows_i, :] += jnp.dot(
                        stripe[:, cw * c:cw * (c + 1)],
                        t1_ref[cw * c:cw * (c + 1), :],
                        preferred_element_type=jnp.float32)

        @pl.when(jnp.logical_and(g >= nmain, g < nmain + nres))
        def _residual_sweep():
            i = bi[g]
            c = bc[g]
            rows_i = pl.ds(i * bsz, bsz)
            t2a_ref[rows_i, :] += jnp.dot(
                adjb_ref[...], t1_ref[pl.ds(c * cw, cw), :],
                preferred_element_type=jnp.float32)

        @pl.when(g >= nmain + nres)
        def _last_chunk_and_finalize():
            i = bi[g]
            rows_i = pl.ds(i * bsz, bsz)
            t2a_ref[rows_i, :] += jnp.dot(
                adjb_ref[:, :lw], t1_ref[(nc - 1) * cw:(nc - 1) * cw + lw, :],
                preferred_element_type=jnp.float32)
            out_ref[...] = p_ref[rows_i, :] + jnp.dot(
                t2a_ref[rows_i, :], 2.0 * w2_ref[...],
                preferred_element_type=jnp.float32)

    return body


def kernel(x, adj, W0, W1, W2, b):
    n, d_in = x.shape
    d_out = W0.shape[1]
    bsz = _pick_block(n)
    nb = n // bsz
    cw = min(_CW, n)
    nc = -(-n // cw)
    lw = n - (nc - 1) * cw
    b2d = b.reshape(1, d_out).astype(jnp.float32)

    ai, bi, bc, oo = [], [], [], []
    for i in range(nb):
        ai.append(i)
        bi.append(0)
        bc.append(0)
        oo.append(0)
    for i in range(nb):
        for c in range(nc - 1):
            if cw * (c + 1) > bsz * i:
                ai.append(nb - 1)
                bi.append(i)
                bc.append(c)
                oo.append(0)
    nres = len(bi) - nb
    for i in range(nb):
        ai.append(nb - 1)
        bi.append(i)
        bc.append(nc - 1)
        oo.append(i)
    ai = jnp.asarray(np.array(ai, dtype=np.int32))
    bi = jnp.asarray(np.array(bi, dtype=np.int32))
    bc = jnp.asarray(np.array(bc, dtype=np.int32))
    oo = jnp.asarray(np.array(oo, dtype=np.int32))

    grid_spec = pltpu.PrefetchScalarGridSpec(
        num_scalar_prefetch=4,
        grid=(nb + nres + nb,),
        in_specs=[
            pl.BlockSpec((n, d_in), lambda g, a, i2, c2, o: (0, 0)),
            pl.BlockSpec((bsz, n), lambda g, a, i2, c2, o: (a[g], 0)),
            pl.BlockSpec((bsz, cw), lambda g, a, i2, c2, o: (i2[g], c2[g])),
            pl.BlockSpec((d_in, d_out), lambda g, a, i2, c2, o: (0, 0)),
            pl.BlockSpec((d_in, d_out), lambda g, a, i2, c2, o: (0, 0)),
            pl.BlockSpec((d_in, d_out), lambda g, a, i2, c2, o: (0, 0)),
            pl.BlockSpec((1, d_out), lambda g, a, i2, c2, o: (0, 0)),
        ],
        out_specs=pl.BlockSpec((bsz, d_out), lambda g, a, i2, c2, o: (o[g], 0)),
        scratch_shapes=[
            pltpu.VMEM((n, d_in), jnp.float32),
            pltpu.VMEM((n, d_out), jnp.float32),
            pltpu.VMEM((n, d_out), jnp.float32),
        ],
    )
    out = pl.pallas_call(
        _make_body(nb, bsz, n, nc, lw, nres, cw),
        grid_spec=grid_spec,
        out_shape=jax.ShapeDtypeStruct((n, d_out), jnp.float32),
        compiler_params=pltpu.CompilerParams(
            dimension_semantics=("arbitrary",),
            vmem_limit_bytes=100 * 1024 * 1024,
        ),
    )(ai, bi, bc, oo, x, adj, adj, W0, W1, W2, b2d)
    return out


# speedup vs baseline: 12.4790x; 1.5274x over previous
import numpy as np
import jax
import jax.numpy as jnp
from jax.experimental import pallas as pl
from jax.experimental.pallas import tpu as pltpu

_CW = 1024
_SUP = 1000


def _pick_block(n):
    for bm in (200, 128, 80, 40, 16, 8):
        if n % bm == 0:
            return bm
    return 1


def _make_body(nb, bsz, n, nc, lw, nres, cw, sup):
    nmain = nb
    spr = max(sup // bsz, 1)

    def body(ai, bi, bc, oo, x_ref, adja_ref, adjb_ref,
             w0_ref, w1_ref, w2_ref, b_ref,
             out_ref, t1_ref, t2a_ref):
        g = pl.program_id(0)

        @pl.when(g < nmain)
        def _main_sweep():
            i = g
            rows_i = pl.ds(i * bsz, bsz)
            stripe = adja_ref[...]
            t1c = jnp.dot(stripe, x_ref[...],
                          preferred_element_type=jnp.float32)
            t1_ref[rows_i, :] = t1c
            t2a_ref[rows_i, :] = jnp.zeros_like(t1c)
            served_rows = sup * (i // spr)
            for c in range(nc - 1):
                @pl.when(cw * (c + 1) <= served_rows)
                def _dual_serve(c=c):
                    t2a_ref[rows_i, :] += jnp.dot(
                        stripe[:, cw * c:cw * (c + 1)],
                        t1_ref[cw * c:cw * (c + 1), :],
                        preferred_element_type=jnp.float32)

        @pl.when(jnp.logical_and(g >= nmain, g < nmain + nres))
        def _residual_sweep():
            r = bi[g]
            c = bc[g]
            rows_r = pl.ds(r * sup, sup)
            t2a_ref[rows_r, :] += jnp.dot(
                adjb_ref[...], t1_ref[pl.ds(c * cw, cw), :],
                preferred_element_type=jnp.float32)

        @pl.when(g >= nmain + nres)
        def _last_chunk_and_finalize():
            r = bi[g]
            rows_r = pl.ds(r * sup, sup)
            t2a_ref[rows_r, :] += jnp.dot(
                adjb_ref[:, :lw], t1_ref[(nc - 1) * cw:(nc - 1) * cw + lw, :],
                preferred_element_type=jnp.float32)
            out_ref[...] = (
                jnp.dot(x_ref[rows_r, :], w0_ref[...] - w2_ref[...],
                        preferred_element_type=jnp.float32)
                + jnp.dot(t1_ref[rows_r, :], w1_ref[...],
                          preferred_element_type=jnp.float32)
                + jnp.dot(t2a_ref[rows_r, :], 2.0 * w2_ref[...],
                          preferred_element_type=jnp.float32)
                + b_ref[...]
            )

    return body


def kernel(x, adj, W0, W1, W2, b):
    n, d_in = x.shape
    d_out = W0.shape[1]
    bsz = _pick_block(n)
    nb = n // bsz
    cw = min(_CW, n)
    sup = _SUP if (n % _SUP == 0 and _SUP % bsz == 0) else bsz
    nsup = n // sup
    spr = sup // bsz
    nc = -(-n // cw)
    lw = n - (nc - 1) * cw
    b2d = b.reshape(1, d_out).astype(jnp.float32)

    ai, bi, bc, oo = [], [], [], []
    for i in range(nb):
        ai.append(i)
        bi.append(0)
        bc.append(0)
        oo.append(0)
    for r in range(nsup):
        for c in range(nc - 1):
            if cw * (c + 1) > sup * r:
                ai.append(nb - 1)
                bi.append(r)
                bc.append(c)
                oo.append(0)
    nres = len(bi) - nb
    for r in range(nsup):
        ai.append(nb - 1)
        bi.append(r)
        bc.append(nc - 1)
        oo.append(r)
    ai = jnp.asarray(np.array(ai, dtype=np.int32))
    bi = jnp.asarray(np.array(bi, dtype=np.int32))
    bc = jnp.asarray(np.array(bc, dtype=np.int32))
    oo = jnp.asarray(np.array(oo, dtype=np.int32))

    grid_spec = pltpu.PrefetchScalarGridSpec(
        num_scalar_prefetch=4,
        grid=(nb + nres + nsup,),
        in_specs=[
            pl.BlockSpec((n, d_in), lambda g, a, i2, c2, o: (0, 0)),
            pl.BlockSpec((bsz, n), lambda g, a, i2, c2, o: (a[g], 0)),
            pl.BlockSpec((sup, cw), lambda g, a, i2, c2, o: (i2[g], c2[g])),
            pl.BlockSpec((d_in, d_out), lambda g, a, i2, c2, o: (0, 0)),
            pl.BlockSpec((d_in, d_out), lambda g, a, i2, c2, o: (0, 0)),
            pl.BlockSpec((d_in, d_out), lambda g, a, i2, c2, o: (0, 0)),
            pl.BlockSpec((1, d_out), lambda g, a, i2, c2, o: (0, 0)),
        ],
        out_specs=pl.BlockSpec((sup, d_out), lambda g, a, i2, c2, o: (o[g], 0)),
        scratch_shapes=[
            pltpu.VMEM((n, d_in), jnp.float32),
            pltpu.VMEM((n, d_out), jnp.float32),
        ],
    )
    out = pl.pallas_call(
        _make_body(nb, bsz, n, nc, lw, nres, cw, sup),
        grid_spec=grid_spec,
        out_shape=jax.ShapeDtypeStruct((n, d_out), jnp.float32),
        compiler_params=pltpu.CompilerParams(
            dimension_semantics=("arbitrary",),
            vmem_limit_bytes=100 * 1024 * 1024,
        ),
    )(ai, bi, bc, oo, x, adj, adj, W0, W1, W2, b2d)
    return out


# speedup vs baseline: 13.8742x; 1.1118x over previous
import numpy as np
import jax
import jax.numpy as jnp
from jax.experimental import pallas as pl
from jax.experimental.pallas import tpu as pltpu

_CW = 2048
_SUP = 1000


def _pick_block(n):
    for bm in (200, 128, 80, 40, 16, 8):
        if n % bm == 0:
            return bm
    return 1


def _make_body(nb, bsz, n, nc, lw, nres, cw, sup):
    nmain = nb
    spr = max(sup // bsz, 1)

    def body(ai, bi, bc, oo, x_ref, adja_ref, adjb_ref,
             w0_ref, w1_ref, w2_ref, b_ref,
             out_ref, t1_ref, t2a_ref):
        g = pl.program_id(0)

        @pl.when(g < nmain)
        def _main_sweep():
            i = g
            rows_i = pl.ds(i * bsz, bsz)
            stripe = adja_ref[...]
            t1c = jnp.dot(stripe, x_ref[...],
                          preferred_element_type=jnp.float32)
            t1_ref[rows_i, :] = t1c
            t2a_ref[rows_i, :] = jnp.zeros_like(t1c)
            served_rows = sup * (i // spr)
            for c in range(nc - 1):
                @pl.when(cw * (c + 1) <= served_rows)
                def _dual_serve(c=c):
                    t2a_ref[rows_i, :] += jnp.dot(
                        stripe[:, cw * c:cw * (c + 1)],
                        t1_ref[cw * c:cw * (c + 1), :],
                        preferred_element_type=jnp.float32)

        @pl.when(jnp.logical_and(g >= nmain, g < nmain + nres))
        def _residual_sweep():
            r = bi[g]
            c = bc[g]
            rows_r = pl.ds(r * sup, sup)
            t2a_ref[rows_r, :] += jnp.dot(
                adjb_ref[...], t1_ref[pl.ds(c * cw, cw), :],
                preferred_element_type=jnp.float32)

        @pl.when(g >= nmain + nres)
        def _last_chunk_and_finalize():
            r = bi[g]
            rows_r = pl.ds(r * sup, sup)
            t2a_ref[rows_r, :] += jnp.dot(
                adjb_ref[:, :lw], t1_ref[(nc - 1) * cw:(nc - 1) * cw + lw, :],
                preferred_element_type=jnp.float32)
            out_ref[...] = (
                jnp.dot(x_ref[rows_r, :], w0_ref[...] - w2_ref[...],
                        preferred_element_type=jnp.float32)
                + jnp.dot(t1_ref[rows_r, :], w1_ref[...],
                          preferred_element_type=jnp.float32)
                + jnp.dot(t2a_ref[rows_r, :], 2.0 * w2_ref[...],
                          preferred_element_type=jnp.float32)
                + b_ref[...]
            )

    return body


def kernel(x, adj, W0, W1, W2, b):
    n, d_in = x.shape
    d_out = W0.shape[1]
    bsz = _pick_block(n)
    nb = n // bsz
    cw = min(_CW, n)
    sup = _SUP if (n % _SUP == 0 and _SUP % bsz == 0) else bsz
    nsup = n // sup
    spr = sup // bsz
    nc = -(-n // cw)
    lw = n - (nc - 1) * cw
    b2d = b.reshape(1, d_out).astype(jnp.float32)

    ai, bi, bc, oo = [], [], [], []
    for i in range(nb):
        ai.append(i)
        bi.append(0)
        bc.append(0)
        oo.append(0)
    for r in range(nsup):
        for c in range(nc - 1):
            if cw * (c + 1) > sup * r:
                ai.append(nb - 1)
                bi.append(r)
                bc.append(c)
                oo.append(0)
    nres = len(bi) - nb
    for r in range(nsup):
        ai.append(nb - 1)
        bi.append(r)
        bc.append(nc - 1)
        oo.append(r)
    ai = jnp.asarray(np.array(ai, dtype=np.int32))
    bi = jnp.asarray(np.array(bi, dtype=np.int32))
    bc = jnp.asarray(np.array(bc, dtype=np.int32))
    oo = jnp.asarray(np.array(oo, dtype=np.int32))

    grid_spec = pltpu.PrefetchScalarGridSpec(
        num_scalar_prefetch=4,
        grid=(nb + nres + nsup,),
        in_specs=[
            pl.BlockSpec((n, d_in), lambda g, a, i2, c2, o: (0, 0)),
            pl.BlockSpec((bsz, n), lambda g, a, i2, c2, o: (a[g], 0)),
            pl.BlockSpec((sup, cw), lambda g, a, i2, c2, o: (i2[g], c2[g])),
            pl.BlockSpec((d_in, d_out), lambda g, a, i2, c2, o: (0, 0)),
            pl.BlockSpec((d_in, d_out), lambda g, a, i2, c2, o: (0, 0)),
            pl.BlockSpec((d_in, d_out), lambda g, a, i2, c2, o: (0, 0)),
            pl.BlockSpec((1, d_out), lambda g, a, i2, c2, o: (0, 0)),
        ],
        out_specs=pl.BlockSpec((sup, d_out), lambda g, a, i2, c2, o: (o[g], 0)),
        scratch_shapes=[
            pltpu.VMEM((n, d_in), jnp.float32),
            pltpu.VMEM((n, d_out), jnp.float32),
        ],
    )
    out = pl.pallas_call(
        _make_body(nb, bsz, n, nc, lw, nres, cw, sup),
        grid_spec=grid_spec,
        out_shape=jax.ShapeDtypeStruct((n, d_out), jnp.float32),
        compiler_params=pltpu.CompilerParams(
            dimension_semantics=("arbitrary",),
            vmem_limit_bytes=100 * 1024 * 1024,
        ),
    )(ai, bi, bc, oo, x, adj, adj, W0, W1, W2, b2d)
    return out
